# baseline (speedup 1.0000x reference)
import functools

import jax
import jax.numpy as jnp
from jax.experimental import pallas as pl
from jax.experimental.pallas import tpu as pltpu

_B = 1048576
_R = _B // 128
_BLK = 256
_STEPS = _R // _BLK

_EPS = 1e-12


def _loss_kernel(lab_ref, plab_ref, go_ref, po_ref, gl_ref, pll_ref,
                 out_ref, acc_ref):
    i = pl.program_id(0)

    label = lab_ref[...] - 2
    t = label.astype(jnp.float32)
    mask_cls = (label >= 0).astype(jnp.float32)
    mask_box = (label != 0).astype(jnp.float32)
    mask_lmk = (label == -2).astype(jnp.float32)

    p = jnp.clip(plab_ref[...], _EPS, 1.0 - _EPS)
    bce = -(t * jnp.log(p) + (1.0 - t) * jnp.log(1.0 - p))
    s_bce = jnp.sum(mask_cls * bce)
    n_cls = jnp.sum(mask_cls)
    n_box = jnp.sum(mask_box)
    n_lmk = jnp.sum(mask_lmk)

    d_o = po_ref[...] - go_ref[...]
    sq_o = d_o * d_o
    d_l = pll_ref[...] - gl_ref[...]
    sq_l = d_l * d_l

    r4 = jax.lax.broadcasted_iota(jnp.int32, (512, 128), 0) // 4
    c4 = jax.lax.broadcasted_iota(jnp.int32, (512, 128), 1)
    g4 = (r4 == c4).astype(jnp.float32)
    rows_o = jax.lax.dot(sq_o, g4, precision=jax.lax.Precision.HIGHEST)

    r10 = jax.lax.broadcasted_iota(jnp.int32, (1280, 128), 0) // 10
    c10 = jax.lax.broadcasted_iota(jnp.int32, (1280, 128), 1)
    g10 = (r10 == c10).astype(jnp.float32)
    rows_l = jax.lax.dot(sq_l, g10, precision=jax.lax.Precision.HIGHEST)

    s_box = jnp.sum(rows_o * mask_box)
    s_lmk = jnp.sum(rows_l * mask_lmk)

    @pl.when(i == 0)
    def _init():
        for k in range(6):
            acc_ref[k] = 0.0

    acc_ref[0] += s_bce
    acc_ref[1] += n_cls
    acc_ref[2] += s_box
    acc_ref[3] += n_box
    acc_ref[4] += s_lmk
    acc_ref[5] += n_lmk

    @pl.when(i == _STEPS - 1)
    def _fin():
        cls_loss = acc_ref[0] / acc_ref[1]
        box_loss = acc_ref[2] / (acc_ref[3] * 4.0)
        lmk_loss = acc_ref[4] / (acc_ref[5] * 10.0)
        total = cls_loss + box_loss + lmk_loss
        out_ref[...] = jnp.full((1, 1), total, dtype=jnp.float32)


@functools.partial(jax.jit)
def kernel(gt_label, pred_label, gt_offset, pred_offset, gt_landmark,
           pred_landmark):
    lab = gt_label.reshape(_R, 128).astype(jnp.int32)
    plab = pred_label.reshape(_R, 128)
    go = gt_offset.reshape(_R, 512)
    po = pred_offset.reshape(_R, 512)
    gl = gt_landmark.reshape(_R, 1280)
    pll = pred_landmark.reshape(_R, 1280)

    out = pl.pallas_call(
        _loss_kernel,
        grid=(_STEPS,),
        in_specs=[
            pl.BlockSpec((_BLK, 128), lambda i: (i, 0)),
            pl.BlockSpec((_BLK, 128), lambda i: (i, 0)),
            pl.BlockSpec((_BLK, 512), lambda i: (i, 0)),
            pl.BlockSpec((_BLK, 512), lambda i: (i, 0)),
            pl.BlockSpec((_BLK, 1280), lambda i: (i, 0)),
            pl.BlockSpec((_BLK, 1280), lambda i: (i, 0)),
        ],
        out_specs=pl.BlockSpec((1, 1), lambda i: (0, 0)),
        out_shape=jax.ShapeDtypeStruct((1, 1), jnp.float32),
        scratch_shapes=[pltpu.SMEM((8,), jnp.float32)],
    )(lab, plab, go, po, gl, pll)
    return out.reshape(())


# speedup vs baseline: 47.3635x; 47.3635x over previous
import functools

import jax
import jax.numpy as jnp
from jax.experimental import pallas as pl
from jax.experimental.pallas import tpu as pltpu

_B = 1048576
_R = _B // 128
_BLK = 256
_CHUNK = _BLK * 128
_STEPS = _R // _BLK

_EPS = 1e-12


def _loss_kernel(lab_ref, plab_ref, lab1_ref, go_ref, po_ref, glt_ref,
                 plt_ref, out_ref, acc_ref):
    i = pl.program_id(0)

    label = lab_ref[...] - 2
    t = label.astype(jnp.float32)
    mask_cls = (label >= 0).astype(jnp.float32)
    mask_box = (label != 0).astype(jnp.float32)

    p = jnp.clip(plab_ref[...], _EPS, 1.0 - _EPS)
    bce = -(t * jnp.log(p) + (1.0 - t) * jnp.log(1.0 - p))
    s_bce = jnp.sum(mask_cls * bce)
    n_cls = jnp.sum(mask_cls)
    n_box = jnp.sum(mask_box)

    d = po_ref[...] - go_ref[...]
    rs_box = jnp.sum(d * d, axis=1)
    s_box = jnp.sum(mask_box * rs_box)

    lab1 = lab1_ref[...]
    mask_lmk_t = (lab1 == 0).astype(jnp.float32)
    n_lmk = jnp.sum(mask_lmk_t)
    dl = plt_ref[...] - glt_ref[...]
    rs_lmk = jnp.sum(dl * dl, axis=0)
    s_lmk = jnp.sum(mask_lmk_t * rs_lmk)

    @pl.when(i == 0)
    def _init():
        for k in range(6):
            acc_ref[k] = 0.0

    acc_ref[0] += s_bce
    acc_ref[1] += n_cls
    acc_ref[2] += s_box
    acc_ref[3] += n_box
    acc_ref[4] += s_lmk
    acc_ref[5] += n_lmk

    @pl.when(i == _STEPS - 1)
    def _fin():
        cls_loss = acc_ref[0] / acc_ref[1]
        box_loss = acc_ref[2] / (acc_ref[3] * 4.0)
        lmk_loss = acc_ref[4] / (acc_ref[5] * 10.0)
        total = cls_loss + box_loss + lmk_loss
        out_ref[...] = jnp.full((1, 1), total, dtype=jnp.float32)


def _native_view(x, c):
    return x.reshape(_R, 128, c).transpose(0, 2, 1)


@functools.partial(jax.jit)
def kernel(gt_label, pred_label, gt_offset, pred_offset, gt_landmark,
           pred_landmark):
    lab32 = gt_label.astype(jnp.int32)
    lab = lab32.reshape(_R, 128)
    plab = pred_label.reshape(_R, 128)
    go = _native_view(gt_offset, 4)
    po = _native_view(pred_offset, 4)
    glt = gt_landmark.T
    plt = pred_landmark.T

    out = pl.pallas_call(
        _loss_kernel,
        grid=(_STEPS,),
        in_specs=[
            pl.BlockSpec((_BLK, 128), lambda i: (i, 0)),
            pl.BlockSpec((_BLK, 128), lambda i: (i, 0)),
            pl.BlockSpec((_CHUNK,), lambda i: (i,)),
            pl.BlockSpec((_BLK, 4, 128), lambda i: (i, 0, 0)),
            pl.BlockSpec((_BLK, 4, 128), lambda i: (i, 0, 0)),
            pl.BlockSpec((10, _CHUNK), lambda i: (0, i)),
            pl.BlockSpec((10, _CHUNK), lambda i: (0, i)),
        ],
        out_specs=pl.BlockSpec((1, 1), lambda i: (0, 0)),
        out_shape=jax.ShapeDtypeStruct((1, 1), jnp.float32),
        scratch_shapes=[pltpu.SMEM((8,), jnp.float32)],
    )(lab, plab, lab32, go, po, glt, plt)
    return out.reshape(())
